# Initial kernel scaffold; baseline (speedup 1.0000x reference)
import jax
import jax.numpy as jnp
from jax.experimental import pallas as pl


def kernel(input_geom, latent_queries, output_queries, x, params):
    raise NotImplementedError("write your pallas kernel here")


# speedup vs baseline: 2.8047x; 2.8047x over previous
import functools

import numpy as np
import jax
import jax.numpy as jnp
from jax import lax
from jax.experimental import pallas as pl
from jax.experimental.pallas import tpu as pltpu
from jax.experimental.pallas import tpu_sc as plsc

_K_IN = 32
_K_OUT = 16
_RADIUS = 0.2
_MODES = 4
_GS = 16
_NLAT = _GS ** 3
_C_FNO = 64
_BIG = 3.0e38


def _build_dft_bases():
    g = _GS
    kx = np.array([0, 1, 2, 3, g - 4, g - 3, g - 2, g - 1])
    kz = np.arange(_MODES)
    xs = np.arange(g)
    ex = np.exp(-2j * np.pi * np.outer(kx, xs) / g)
    ez = np.exp(-2j * np.pi * np.outer(kz, xs) / g)
    E = (ex[:, None, None, :, None, None]
         * ex[None, :, None, None, :, None]
         * ez[None, None, :, None, None, :])
    E = E.reshape(256, g ** 3)
    half = g // 2 + 1
    onehot = np.zeros((256, g, g, half), np.complex128)
    mi = 0
    for ix in range(8):
        for iy in range(8):
            for iz in range(_MODES):
                onehot[mi, kx[ix], kx[iy], kz[iz]] = 1.0
                mi += 1
    A = np.fft.irfftn(onehot, s=(g, g, g), axes=(1, 2, 3)).reshape(256, -1)
    B = np.fft.irfftn(1j * onehot, s=(g, g, g), axes=(1, 2, 3)).reshape(256, -1)
    return (E.real.astype(np.float32), E.imag.astype(np.float32),
            A.T.astype(np.float32).copy(), B.T.astype(np.float32).copy())


_ER_NP, _EI_NP, _GTA_NP, _GTB_NP = _build_dft_bases()


def _gelu(x):
    return jax.nn.gelu(x)


def _knn_body(q_ref, p_ref, idx_ref, d2_ref, scratch, *, k, np_, qb, cw):
    q = q_ref[...]
    qn = jnp.sum(q * q, axis=1, keepdims=True)
    for c0 in range(0, np_, cw):
        pc = p_ref[:, c0:c0 + cw]
        pn = jnp.sum(pc * pc, axis=0, keepdims=True)
        dot = jnp.dot(q, pc, preferred_element_type=jnp.float32)
        scratch[:, c0:c0 + cw] = qn + pn - 2.0 * dot
    iota = lax.broadcasted_iota(jnp.int32, (qb, np_), 1)
    kio = lax.broadcasted_iota(jnp.int32, (qb, k), 1)

    def step(j, carry):
        acc_i, acc_d, pm, pji = carry
        d2 = scratch[...]
        valid = (d2 > pm) | ((d2 == pm) & (iota > pji))
        dm = jnp.where(valid, d2, _BIG)
        m = jnp.min(dm, axis=1, keepdims=True)
        ji = jnp.min(jnp.where((dm == m) & valid, iota, np_),
                     axis=1, keepdims=True)
        acc_i = jnp.where(kio == j, ji, acc_i)
        acc_d = jnp.where(kio == j, m, acc_d)
        return acc_i, acc_d, m, ji

    init = (jnp.zeros((qb, k), jnp.int32), jnp.zeros((qb, k), jnp.float32),
            jnp.full((qb, 1), -1.0, jnp.float32),
            jnp.full((qb, 1), -1, jnp.int32))
    acc_i, acc_d, _, _ = lax.fori_loop(0, k, step, init)
    idx_ref[...] = acc_i
    d2_ref[...] = acc_d


def _knn(qpad, pT, k, qb=256, cw=2048):
    Qp = qpad.shape[0]
    Np = pT.shape[1]
    body = functools.partial(_knn_body, k=k, np_=Np, qb=qb, cw=min(cw, Np))
    return pl.pallas_call(
        body,
        grid=(Qp // qb,),
        in_specs=[pl.BlockSpec((qb, 128), lambda i: (i, 0)),
                  pl.BlockSpec((128, Np), lambda i: (0, 0))],
        out_specs=[pl.BlockSpec((qb, k), lambda i: (i, 0)),
                   pl.BlockSpec((qb, k), lambda i: (i, 0))],
        out_shape=[jax.ShapeDtypeStruct((Qp, k), jnp.int32),
                   jax.ShapeDtypeStruct((Qp, k), jnp.float32)],
        scratch_shapes=[pltpu.VMEM((qb, Np), jnp.float32)],
    )(qpad, pT)


def _sc_gather(table, idx3):
    NW, nch, lw = idx3.shape
    D = table.shape[1]
    B = NW * nch * lw
    info = plsc.get_sparse_core_info()
    NC = info.num_cores
    mesh = plsc.VectorSubcoreMesh(core_axis_name="c", subcore_axis_name="s")

    @functools.partial(
        pl.kernel, mesh=mesh,
        compiler_params=pltpu.CompilerParams(use_tc_tiling_on_sc=False),
        out_type=jax.ShapeDtypeStruct((B, D), jnp.float32),
        scratch_types=[pltpu.VMEM((nch, lw), jnp.int32),
                       pltpu.VMEM((lw, D), jnp.float32),
                       pltpu.VMEM((lw, D), jnp.float32),
                       pltpu.SemaphoreType.DMA,
                       pltpu.SemaphoreType.DMA],
    )
    def gather_k(table_hbm, idx_hbm, out_hbm, idx_v, buf0, buf1, sem0, sem1):
        wid = lax.axis_index("s") * NC + lax.axis_index("c")
        pltpu.sync_copy(idx_hbm.at[wid], idx_v)
        base = wid * (nch * lw)

        def chunk(c, carry):
            pltpu.async_copy(table_hbm.at[idx_v.at[c]], buf0, sem0).wait()
            pltpu.sync_copy(buf0, out_hbm.at[pl.ds(base + c * lw, lw)])
            return carry

        lax.fori_loop(0, nch, chunk, 0)

    return gather_k(table, idx3)


def _gno_body(q_ref, g_ref, d2_ref, *refs, k, qb, cfeat, r2, nrest):
    wq_ref, wp_ref, b1_ref = refs[0], refs[1], refs[2]
    rest = refs[3:3 + 2 * nrest]
    ew1_ref, eb1_ref, ew2_ref, eb2_ref, out_ref = refs[3 + 2 * nrest:]
    q = q_ref[...]
    g = g_ref[...]
    h1 = wq_ref.shape[1]
    qpart = jnp.dot(q, wq_ref[...], preferred_element_type=jnp.float32)
    h = jnp.dot(g, wp_ref[...], preferred_element_type=jnp.float32)
    h = h + jnp.broadcast_to(qpart[:, None, :], (qb, k, h1)).reshape(qb * k, h1)
    h = _gelu(h + b1_ref[...])
    for i in range(nrest):
        w, b = rest[2 * i][...], rest[2 * i + 1][...]
        h = jnp.dot(h, w, preferred_element_type=jnp.float32) + b
        if i < nrest - 1:
            h = _gelu(h)
    feats = g[:, 3:3 + cfeat]
    rep = (h * feats).reshape(qb, k, cfeat)
    mask = (d2_ref[...] <= r2).astype(jnp.float32)
    acc = jnp.zeros((qb, cfeat), jnp.float32)
    for j in range(k):
        acc = acc + rep[:, j, :] * mask[:, j:j + 1]
    denom = jnp.maximum(jnp.sum(mask, axis=1, keepdims=True), 1.0)
    o = acc / denom
    e = _gelu(jnp.dot(o, ew1_ref[...], preferred_element_type=jnp.float32)
              + eb1_ref[...])
    out_ref[...] = (jnp.dot(e, ew2_ref[...], preferred_element_type=jnp.float32)
                    + eb2_ref[...])


def _gno_mlp(qpad, g, d2s, mlp_p, epi_p, k, cfeat, qb=128):
    Qp = qpad.shape[0]
    Dpad = g.shape[1]
    Ws, bs = mlp_p["Ws"], mlp_p["bs"]
    w1 = Ws[0]
    h1 = w1.shape[1]
    wq = jnp.zeros((128, h1), jnp.float32).at[:3].set(w1[:3])
    wp = jnp.zeros((Dpad, h1), jnp.float32).at[:3].set(w1[3:6])
    ops = [qpad, g, d2s, wq, wp, bs[0].reshape(1, -1)]
    for w, b in zip(Ws[1:], bs[1:]):
        ops += [w, b.reshape(1, -1)]
    eWs, ebs = epi_p["Ws"], epi_p["bs"]
    ops += [eWs[0], ebs[0].reshape(1, -1), eWs[1], ebs[1].reshape(1, -1)]
    cout = eWs[1].shape[1]
    nrest = len(Ws) - 1

    in_specs = [pl.BlockSpec((qb, 128), lambda i: (i, 0)),
                pl.BlockSpec((qb * k, Dpad), lambda i: (i, 0)),
                pl.BlockSpec((qb, k), lambda i: (i, 0))]
    for op in ops[3:]:
        in_specs.append(pl.BlockSpec(op.shape, lambda i: (0, 0)))

    body = functools.partial(_gno_body, k=k, qb=qb, cfeat=cfeat,
                             r2=_RADIUS * _RADIUS, nrest=nrest)
    return pl.pallas_call(
        body,
        grid=(Qp // qb,),
        in_specs=in_specs,
        out_specs=pl.BlockSpec((qb, cout), lambda i: (i, 0)),
        out_shape=jax.ShapeDtypeStruct((Qp, cout), jnp.float32),
    )(*ops)


def _fno_body(h_ref, er_ref, ei_ref, ga_ref, gb_ref, wmr_ref, wmi_ref,
              sw_ref, sb_ref, out_ref, xrt_s, xit_s, accr_s, acci_s,
              *, nlayers, c):
    l = pl.program_id(0)
    h = jnp.where(l == 0, h_ref[...], out_ref[...])
    xr = jnp.dot(er_ref[...], h, preferred_element_type=jnp.float32)
    xi = jnp.dot(ei_ref[...], h, preferred_element_type=jnp.float32)
    xrt_s[...] = xr.T
    xit_s[...] = xi.T
    accr_s[...] = jnp.zeros((c, 256), jnp.float32)
    acci_s[...] = jnp.zeros((c, 256), jnp.float32)

    def mix(i, carry):
        a = xrt_s[pl.ds(i, 1), :]
        b = xit_s[pl.ds(i, 1), :]
        wr = wmr_ref[0, i]
        wi = wmi_ref[0, i]
        accr_s[...] = accr_s[...] + a * wr - b * wi
        acci_s[...] = acci_s[...] + a * wi + b * wr
        return carry

    lax.fori_loop(0, c, mix, 0)
    orr = accr_s[...].T
    oi = acci_s[...].T
    conv = (jnp.dot(ga_ref[...], orr, preferred_element_type=jnp.float32)
            + jnp.dot(gb_ref[...], oi, preferred_element_type=jnp.float32))
    skip = (jnp.dot(h, sw_ref[0], preferred_element_type=jnp.float32)
            + sb_ref[0])
    h = conv + skip
    out_ref[...] = jnp.where(l < nlayers - 1, _gelu(h), h)


def _fno(h, fno_params):
    c = _C_FNO
    nl = len(fno_params)
    wmr, wmi, sws, sbs = [], [], [], []
    for lp in fno_params:
        w4r = lp["wr"].reshape(2, 2, c, c, _MODES, _MODES, _MODES)
        w4i = lp["wi"].reshape(2, 2, c, c, _MODES, _MODES, _MODES)
        wmr.append(jnp.transpose(w4r, (2, 3, 0, 4, 1, 5, 6)).reshape(c, c, 256))
        wmi.append(jnp.transpose(w4i, (2, 3, 0, 4, 1, 5, 6)).reshape(c, c, 256))
        sws.append(lp["skip_W"])
        sbs.append(lp["skip_b"].reshape(1, c))
    wmr = jnp.stack(wmr)
    wmi = jnp.stack(wmi)
    sw = jnp.stack(sws)
    sb = jnp.stack(sbs)
    ops = [h, jnp.asarray(_ER_NP), jnp.asarray(_EI_NP),
           jnp.asarray(_GTA_NP), jnp.asarray(_GTB_NP), wmr, wmi, sw, sb]
    in_specs = [pl.BlockSpec(h.shape, lambda l: (0, 0)),
                pl.BlockSpec(_ER_NP.shape, lambda l: (0, 0)),
                pl.BlockSpec(_EI_NP.shape, lambda l: (0, 0)),
                pl.BlockSpec(_GTA_NP.shape, lambda l: (0, 0)),
                pl.BlockSpec(_GTB_NP.shape, lambda l: (0, 0)),
                pl.BlockSpec((1, c, c, 256), lambda l: (l, 0, 0, 0)),
                pl.BlockSpec((1, c, c, 256), lambda l: (l, 0, 0, 0)),
                pl.BlockSpec((1, c, c), lambda l: (l, 0, 0)),
                pl.BlockSpec((1, 1, c), lambda l: (l, 0, 0))]
    body = functools.partial(_fno_body, nlayers=nl, c=c)
    return pl.pallas_call(
        body,
        grid=(nl,),
        in_specs=in_specs,
        out_specs=pl.BlockSpec((_NLAT, c), lambda l: (0, 0)),
        out_shape=jax.ShapeDtypeStruct((_NLAT, c), jnp.float32),
        scratch_shapes=[pltpu.VMEM((c, 256), jnp.float32)] * 4,
    )(*ops)


def _pad_cols(a, w=128):
    return jnp.pad(a, ((0, 0), (0, w - a.shape[1])))


def _pad_pT(p, np_pad):
    n = p.shape[0]
    pT = jnp.full((3, np_pad), 100.0, jnp.float32).at[:, :n].set(p.T)
    return jnp.pad(pT, ((0, 125), (0, 0)))


def _table(coords, feats, dpad):
    t = jnp.concatenate([coords, feats], axis=1)
    return jnp.pad(t, ((0, 0), (0, dpad - t.shape[1])))


def kernel(input_geom, latent_queries, output_queries, x, params):
    lat = latent_queries.reshape(-1, 3)

    q1 = _pad_cols(lat)
    pT1 = _pad_pT(input_geom, 10240)
    idx1, d21 = _knn(q1, pT1, _K_IN)
    t1 = _table(input_geom, x, 144)
    g1 = _sc_gather(t1, idx1.reshape(32, -1, 128))
    h = _gno_mlp(q1, g1, d21, params["in_gno"], params["lifting"],
                 _K_IN, 128)

    lat_feats = _fno(h, params["fno"])

    nq = output_queries.shape[0]
    q2 = _pad_cols(jnp.pad(output_queries, ((0, 10240 - nq), (0, 0))))
    pT2 = _pad_pT(lat, _NLAT)
    idx2, d22 = _knn(q2, pT2, _K_OUT)
    t2 = _table(lat, lat_feats, 80)
    g2 = _sc_gather(t2, idx2.reshape(32, -1, 128))
    out = _gno_mlp(q2, g2, d22, params["out_gno"], params["projection"],
                   _K_OUT, 64)
    return out[:nq]


# speedup vs baseline: 3.7969x; 1.3537x over previous
import functools

import numpy as np
import jax
import jax.numpy as jnp
from jax import lax
from jax.experimental import pallas as pl
from jax.experimental.pallas import tpu as pltpu
from jax.experimental.pallas import tpu_sc as plsc

_K_IN = 32
_K_OUT = 16
_RADIUS = 0.2
_MODES = 4
_GS = 16
_NLAT = _GS ** 3
_C_FNO = 64
_BIG = 3.0e38


def _build_dft_bases():
    g = _GS
    kx = np.array([0, 1, 2, 3, g - 4, g - 3, g - 2, g - 1])
    kz = np.arange(_MODES)
    xs = np.arange(g)
    ex = np.exp(-2j * np.pi * np.outer(kx, xs) / g)
    ez = np.exp(-2j * np.pi * np.outer(kz, xs) / g)
    E = (ex[:, None, None, :, None, None]
         * ex[None, :, None, None, :, None]
         * ez[None, None, :, None, None, :])
    E = E.reshape(256, g ** 3)
    half = g // 2 + 1
    onehot = np.zeros((256, g, g, half), np.complex128)
    mi = 0
    for ix in range(8):
        for iy in range(8):
            for iz in range(_MODES):
                onehot[mi, kx[ix], kx[iy], kz[iz]] = 1.0
                mi += 1
    A = np.fft.irfftn(onehot, s=(g, g, g), axes=(1, 2, 3)).reshape(256, -1)
    B = np.fft.irfftn(1j * onehot, s=(g, g, g), axes=(1, 2, 3)).reshape(256, -1)
    return (E.real.astype(np.float32), E.imag.astype(np.float32),
            A.T.astype(np.float32).copy(), B.T.astype(np.float32).copy())


_ER_NP, _EI_NP, _GTA_NP, _GTB_NP = _build_dft_bases()


def _gelu(x):
    return jax.nn.gelu(x)


def _knn_body(q_ref, p_ref, idx_ref, d2_ref, scratch, *, k, np_, qb, cw):
    q = q_ref[...]
    qn = jnp.sum(q * q, axis=1, keepdims=True)
    for c0 in range(0, np_, cw):
        pc = p_ref[:, c0:c0 + cw]
        pn = jnp.sum(pc * pc, axis=0, keepdims=True)
        dot = jnp.dot(q, pc, preferred_element_type=jnp.float32)
        scratch[:, c0:c0 + cw] = qn + pn - 2.0 * dot
    iota = lax.broadcasted_iota(jnp.int32, (qb, np_), 1)
    kio = lax.broadcasted_iota(jnp.int32, (qb, k), 1)

    def step(j, carry):
        acc_i, acc_d = carry
        d2 = scratch[...]
        m = jnp.min(d2, axis=1, keepdims=True)
        ji = jnp.min(jnp.where(d2 == m, iota, np_), axis=1, keepdims=True)
        scratch[...] = jnp.where(iota == ji, _BIG, d2)
        acc_i = jnp.where(kio == j, ji, acc_i)
        acc_d = jnp.where(kio == j, m, acc_d)
        return acc_i, acc_d

    init = (jnp.zeros((qb, k), jnp.int32), jnp.zeros((qb, k), jnp.float32))
    acc_i, acc_d = lax.fori_loop(0, k, step, init)
    idx_ref[...] = acc_i
    d2_ref[...] = acc_d


def _knn(qpad, pT, k, qb=256, cw=2048):
    Qp = qpad.shape[0]
    Np = pT.shape[1]
    body = functools.partial(_knn_body, k=k, np_=Np, qb=qb, cw=min(cw, Np))
    return pl.pallas_call(
        body,
        grid=(Qp // qb,),
        in_specs=[pl.BlockSpec((qb, 128), lambda i: (i, 0)),
                  pl.BlockSpec((128, Np), lambda i: (0, 0))],
        out_specs=[pl.BlockSpec((qb, k), lambda i: (i, 0)),
                   pl.BlockSpec((qb, k), lambda i: (i, 0))],
        out_shape=[jax.ShapeDtypeStruct((Qp, k), jnp.int32),
                   jax.ShapeDtypeStruct((Qp, k), jnp.float32)],
        scratch_shapes=[pltpu.VMEM((qb, Np), jnp.float32)],
    )(qpad, pT)


def _sc_gather(table, idx3):
    NW, nch, lw = idx3.shape
    D = table.shape[1]
    B = NW * nch * lw
    info = plsc.get_sparse_core_info()
    NC = info.num_cores
    mesh = plsc.VectorSubcoreMesh(core_axis_name="c", subcore_axis_name="s")

    @functools.partial(
        pl.kernel, mesh=mesh,
        compiler_params=pltpu.CompilerParams(use_tc_tiling_on_sc=False),
        out_type=jax.ShapeDtypeStruct((B, D), jnp.float32),
        scratch_types=[pltpu.VMEM((nch, lw), jnp.int32),
                       pltpu.VMEM((lw, D), jnp.float32),
                       pltpu.VMEM((lw, D), jnp.float32),
                       pltpu.SemaphoreType.DMA,
                       pltpu.SemaphoreType.DMA],
    )
    def gather_k(table_hbm, idx_hbm, out_hbm, idx_v, buf0, buf1, sem0, sem1):
        wid = lax.axis_index("s") * NC + lax.axis_index("c")
        pltpu.sync_copy(idx_hbm.at[wid], idx_v)
        base = wid * (nch * lw)

        def chunk(c, carry):
            pltpu.async_copy(table_hbm.at[idx_v.at[c]], buf0, sem0).wait()
            pltpu.sync_copy(buf0, out_hbm.at[pl.ds(base + c * lw, lw)])
            return carry

        lax.fori_loop(0, nch, chunk, 0)

    return gather_k(table, idx3)


def _gno_body(q_ref, g_ref, d2_ref, d2f_ref, *refs, k, qb, cfeat, r2, nrest):
    wq_ref, wp_ref, b1_ref = refs[0], refs[1], refs[2]
    rest = refs[3:3 + 2 * nrest]
    s_ref, ew1_ref, eb1_ref, ew2_ref, eb2_ref, out_ref = refs[3 + 2 * nrest:]
    q = q_ref[...]
    g = g_ref[...]
    h1 = wq_ref.shape[1]
    qpart = jnp.dot(q, wq_ref[...], preferred_element_type=jnp.float32)
    h = jnp.dot(g, wp_ref[...], preferred_element_type=jnp.float32)
    h = h + jnp.broadcast_to(qpart[:, None, :], (qb, k, h1)).reshape(qb * k, h1)
    h = _gelu(h + b1_ref[...])
    for i in range(nrest):
        w, b = rest[2 * i][...], rest[2 * i + 1][...]
        h = jnp.dot(h, w, preferred_element_type=jnp.float32) + b
        if i < nrest - 1:
            h = _gelu(h)
    feats = g[:, 3:3 + cfeat]
    maskf = (d2f_ref[...] <= r2).astype(jnp.float32)
    rep = h * feats * maskf
    acc = jnp.dot(s_ref[...], rep, preferred_element_type=jnp.float32)
    mask = (d2_ref[...] <= r2).astype(jnp.float32)
    denom = jnp.maximum(jnp.sum(mask, axis=1, keepdims=True), 1.0)
    o = acc / denom
    e = _gelu(jnp.dot(o, ew1_ref[...], preferred_element_type=jnp.float32)
              + eb1_ref[...])
    out_ref[...] = (jnp.dot(e, ew2_ref[...], preferred_element_type=jnp.float32)
                    + eb2_ref[...])


def _gno_mlp(qpad, g, d2s, mlp_p, epi_p, k, cfeat, qb=128):
    Qp = qpad.shape[0]
    Dpad = g.shape[1]
    Ws, bs = mlp_p["Ws"], mlp_p["bs"]
    w1 = Ws[0]
    h1 = w1.shape[1]
    wq = jnp.zeros((128, h1), jnp.float32).at[:3].set(w1[:3])
    wp = jnp.zeros((Dpad, h1), jnp.float32).at[:3].set(w1[3:6])
    ops = [qpad, g, d2s, d2s.reshape(-1, 1), wq, wp, bs[0].reshape(1, -1)]
    for w, b in zip(Ws[1:], bs[1:]):
        ops += [w, b.reshape(1, -1)]
    smat = jnp.asarray(np.repeat(np.eye(qb, dtype=np.float32), k, axis=1))
    eWs, ebs = epi_p["Ws"], epi_p["bs"]
    ops += [smat, eWs[0], ebs[0].reshape(1, -1), eWs[1], ebs[1].reshape(1, -1)]
    cout = eWs[1].shape[1]
    nrest = len(Ws) - 1

    in_specs = [pl.BlockSpec((qb, 128), lambda i: (i, 0)),
                pl.BlockSpec((qb * k, Dpad), lambda i: (i, 0)),
                pl.BlockSpec((qb, k), lambda i: (i, 0)),
                pl.BlockSpec((qb * k, 1), lambda i: (i, 0))]
    for op in ops[4:]:
        in_specs.append(pl.BlockSpec(op.shape, lambda i: (0, 0)))

    body = functools.partial(_gno_body, k=k, qb=qb, cfeat=cfeat,
                             r2=_RADIUS * _RADIUS, nrest=nrest)
    return pl.pallas_call(
        body,
        grid=(Qp // qb,),
        in_specs=in_specs,
        out_specs=pl.BlockSpec((qb, cout), lambda i: (i, 0)),
        out_shape=jax.ShapeDtypeStruct((Qp, cout), jnp.float32),
    )(*ops)


def _fno_body(h_ref, er_ref, ei_ref, ga_ref, gb_ref, wmr_ref, wmi_ref,
              sw_ref, sb_ref, out_ref, xrt_s, xit_s, accr_s, acci_s,
              *, nlayers, c):
    l = pl.program_id(0)
    h = jnp.where(l == 0, h_ref[...], out_ref[...])
    xr = jnp.dot(er_ref[...], h, preferred_element_type=jnp.float32)
    xi = jnp.dot(ei_ref[...], h, preferred_element_type=jnp.float32)
    xrt_s[...] = xr.T
    xit_s[...] = xi.T
    accr_s[...] = jnp.zeros((c, 256), jnp.float32)
    acci_s[...] = jnp.zeros((c, 256), jnp.float32)

    def mix(i, carry):
        a = xrt_s[pl.ds(i, 1), :]
        b = xit_s[pl.ds(i, 1), :]
        wr = wmr_ref[0, i]
        wi = wmi_ref[0, i]
        accr_s[...] = accr_s[...] + a * wr - b * wi
        acci_s[...] = acci_s[...] + a * wi + b * wr
        return carry

    lax.fori_loop(0, c, mix, 0)
    orr = accr_s[...].T
    oi = acci_s[...].T
    conv = (jnp.dot(ga_ref[...], orr, preferred_element_type=jnp.float32)
            + jnp.dot(gb_ref[...], oi, preferred_element_type=jnp.float32))
    skip = (jnp.dot(h, sw_ref[0], preferred_element_type=jnp.float32)
            + sb_ref[0])
    h = conv + skip
    out_ref[...] = jnp.where(l < nlayers - 1, _gelu(h), h)


def _fno(h, fno_params):
    c = _C_FNO
    nl = len(fno_params)
    wmr, wmi, sws, sbs = [], [], [], []
    for lp in fno_params:
        w4r = lp["wr"].reshape(2, 2, c, c, _MODES, _MODES, _MODES)
        w4i = lp["wi"].reshape(2, 2, c, c, _MODES, _MODES, _MODES)
        wmr.append(jnp.transpose(w4r, (2, 3, 0, 4, 1, 5, 6)).reshape(c, c, 256))
        wmi.append(jnp.transpose(w4i, (2, 3, 0, 4, 1, 5, 6)).reshape(c, c, 256))
        sws.append(lp["skip_W"])
        sbs.append(lp["skip_b"].reshape(1, c))
    wmr = jnp.stack(wmr)
    wmi = jnp.stack(wmi)
    sw = jnp.stack(sws)
    sb = jnp.stack(sbs)
    ops = [h, jnp.asarray(_ER_NP), jnp.asarray(_EI_NP),
           jnp.asarray(_GTA_NP), jnp.asarray(_GTB_NP), wmr, wmi, sw, sb]
    in_specs = [pl.BlockSpec(h.shape, lambda l: (0, 0)),
                pl.BlockSpec(_ER_NP.shape, lambda l: (0, 0)),
                pl.BlockSpec(_EI_NP.shape, lambda l: (0, 0)),
                pl.BlockSpec(_GTA_NP.shape, lambda l: (0, 0)),
                pl.BlockSpec(_GTB_NP.shape, lambda l: (0, 0)),
                pl.BlockSpec((1, c, c, 256), lambda l: (l, 0, 0, 0)),
                pl.BlockSpec((1, c, c, 256), lambda l: (l, 0, 0, 0)),
                pl.BlockSpec((1, c, c), lambda l: (l, 0, 0)),
                pl.BlockSpec((1, 1, c), lambda l: (l, 0, 0))]
    body = functools.partial(_fno_body, nlayers=nl, c=c)
    return pl.pallas_call(
        body,
        grid=(nl,),
        in_specs=in_specs,
        out_specs=pl.BlockSpec((_NLAT, c), lambda l: (0, 0)),
        out_shape=jax.ShapeDtypeStruct((_NLAT, c), jnp.float32),
        scratch_shapes=[pltpu.VMEM((c, 256), jnp.float32)] * 4,
    )(*ops)


def _pad_cols(a, w=128):
    return jnp.pad(a, ((0, 0), (0, w - a.shape[1])))


def _pad_pT(p, np_pad):
    n = p.shape[0]
    pT = jnp.full((3, np_pad), 100.0, jnp.float32).at[:, :n].set(p.T)
    return jnp.pad(pT, ((0, 125), (0, 0)))


def _table(coords, feats, dpad):
    t = jnp.concatenate([coords, feats], axis=1)
    return jnp.pad(t, ((0, 0), (0, dpad - t.shape[1])))


def kernel(input_geom, latent_queries, output_queries, x, params):
    lat = latent_queries.reshape(-1, 3)

    q1 = _pad_cols(lat)
    pT1 = _pad_pT(input_geom, 10240)
    idx1, d21 = _knn(q1, pT1, _K_IN)
    nq = output_queries.shape[0]
    q2 = _pad_cols(jnp.pad(output_queries, ((0, 10240 - nq), (0, 0))))
    pT2 = _pad_pT(lat, _NLAT)
    idx2, d22 = _knn(q2, pT2, _K_OUT)

    t1 = _table(input_geom, x, 144)
    g1 = _sc_gather(t1, idx1.reshape(32, -1, 128))
    h = _gno_mlp(q1, g1, d21, params["in_gno"], params["lifting"],
                 _K_IN, 128)

    lat_feats = _fno(h, params["fno"])

    t2 = _table(lat, lat_feats, 80)
    g2 = _sc_gather(t2, idx2.reshape(32, -1, 128))
    out = _gno_mlp(q2, g2, d22, params["out_gno"], params["projection"],
                   _K_OUT, 64)
    return out[:nq]
